# Initial kernel scaffold; baseline (speedup 1.0000x reference)
import jax
import jax.numpy as jnp
from jax import lax
from jax.experimental import pallas as pl
from jax.experimental.pallas import tpu as pltpu
from jax.experimental.pallas import tpu_sc as plsc

N = 10000
E = 320000
D = 128
DE = 16
G = 256
H = 128

NC = 2
NS = 16
NW = NC * NS
EP = E // NW
CW = 80
NCH = EP // CW
EB = 2000
NB = EP // EB
SB = EB // CW
XCH = N // CW


def _sc_segment_sums(x, col, edge_attr, batch_flat, batch2d):
    mesh = plsc.VectorSubcoreMesh(core_axis_name="c", subcore_axis_name="s")

    def body(x_hbm, col_hbm, ea_hbm, batchf_hbm, batch2_hbm,
             node_out, edge_out,
             acc_node, acc_edge,
             batch_f, batch_2, col_v, seg_v, ebuf, xbuf, zrow, zrow_e):
        cid = lax.axis_index("c")
        sid = lax.axis_index("s")
        wid = cid * NS + sid

        z16 = jnp.zeros((16,), jnp.float32)
        for r in range(16):
            for o in range(D // 16):
                zrow[r, pl.ds(o * 16, 16)] = z16
            zrow_e[r, pl.ds(0, DE)] = z16
        pltpu.sync_copy(zrow, acc_node.at[pl.ds(sid * 16, 16)])
        pltpu.sync_copy(zrow_e, acc_edge.at[pl.ds(sid * 16, 16)])

        pltpu.sync_copy(batchf_hbm, batch_f)
        pltpu.sync_copy(batch2_hbm, batch_2)
        pltpu.sync_copy(col_hbm.at[pl.ds(wid * EP, EP)], col_v)

        def seg_body(r, carry):
            for o in range(CW // 16):
                cv = col_v[pl.ds(pl.multiple_of(r * CW + o * 16, 16), 16)]
                sv = plsc.load_gather(batch_f, [cv])
                seg_v[r, pl.ds(o * 16, 16)] = sv
            return carry
        lax.fori_loop(0, NCH, seg_body, 0)

        plsc.subcore_barrier()

        for b in range(NB):
            pltpu.sync_copy(ea_hbm.at[pl.ds(wid * EP + b * EB, EB)], ebuf)

            def sc_body(s, carry):
                off = pl.multiple_of(s * CW, CW)
                pltpu.sync_copy(ebuf.at[pl.ds(off, CW)],
                                acc_edge.at[seg_v.at[b * SB + s]], add=True)
                return carry
            lax.fori_loop(0, SB, sc_body, 0)

        for k in range((XCH + NW - 1) // NW):
            c = wid + NW * k

            @pl.when(c < XCH)
            def _():
                off = pl.multiple_of(c * CW, CW)
                pltpu.sync_copy(x_hbm.at[pl.ds(off, CW)], xbuf)
                pltpu.sync_copy(xbuf, acc_node.at[batch_2.at[c]], add=True)

        plsc.subcore_barrier()

        pltpu.sync_copy(acc_node.at[pl.ds(sid * 16, 16)],
                        node_out.at[cid, pl.ds(sid * 16, 16)])
        pltpu.sync_copy(acc_edge.at[pl.ds(sid * 16, 16)],
                        edge_out.at[cid, pl.ds(sid * 16, 16)])

    f = pl.kernel(
        body,
        out_type=(jax.ShapeDtypeStruct((NC, G, D), jnp.float32),
                  jax.ShapeDtypeStruct((NC, G, DE), jnp.float32)),
        mesh=mesh,
        scratch_types=[
            pltpu.VMEM_SHARED((G, D), jnp.float32),
            pltpu.VMEM_SHARED((G, DE), jnp.float32),
            pltpu.VMEM((N,), jnp.int32),
            pltpu.VMEM((XCH, CW), jnp.int32),
            pltpu.VMEM((EP,), jnp.int32),
            pltpu.VMEM((NCH, CW), jnp.int32),
            pltpu.VMEM((EB, DE), jnp.float32),
            pltpu.VMEM((CW, D), jnp.float32),
            pltpu.VMEM((16, D), jnp.float32),
            pltpu.VMEM((16, DE), jnp.float32),
        ],
    )
    return f(x, col, edge_attr, batch_flat, batch2d)


def _tc_mlp(node_part, edge_part, W1n, W1e, b1, g1, be1, W2, b2, g2, be2,
            W3, b3):
    def body(np_ref, ep_ref, w1n_ref, w1e_ref, b1_ref, g1_ref, be1_ref,
             w2_ref, b2_ref, g2_ref, be2_ref, w3_ref, b3_ref, o_ref):
        na = np_ref[0] + np_ref[1]
        ea = ep_ref[0] + ep_ref[1]
        h = jnp.dot(na, w1n_ref[...], preferred_element_type=jnp.float32)
        h = h + jnp.dot(ea, w1e_ref[...], preferred_element_type=jnp.float32)
        h = h + b1_ref[...]
        h = jnp.where(h >= 0, h, 0.01 * h)
        mean = jnp.mean(h, axis=0, keepdims=True)
        var = jnp.mean((h - mean) ** 2, axis=0, keepdims=True)
        h = (h - mean) / jnp.sqrt(var + 1e-5) * g1_ref[...] + be1_ref[...]
        h = jnp.dot(h, w2_ref[...], preferred_element_type=jnp.float32)
        h = h + b2_ref[...]
        h = jnp.where(h >= 0, h, 0.01 * h)
        mean = jnp.mean(h, axis=0, keepdims=True)
        var = jnp.mean((h - mean) ** 2, axis=0, keepdims=True)
        h = (h - mean) / jnp.sqrt(var + 1e-5) * g2_ref[...] + be2_ref[...]
        h = jnp.dot(h, w3_ref[...], preferred_element_type=jnp.float32)
        o_ref[...] = h + b3_ref[...]

    return pl.pallas_call(
        body,
        out_shape=jax.ShapeDtypeStruct((G, H), jnp.float32),
    )(node_part, edge_part, W1n, W1e, b1, g1, be1, W2, b2, g2, be2, W3, b3)


def kernel(x, edge_index, edge_attr, u, batch, W1, b1, g1, be1, W2, b2, g2,
           be2, W3, b3):
    del u
    col = edge_index[1]
    batch2d = batch.reshape(XCH, CW)
    node_part, edge_part = _sc_segment_sums(x, col, edge_attr, batch, batch2d)
    return _tc_mlp(node_part, edge_part,
                   W1[:D], W1[D:],
                   b1.reshape(1, H), g1.reshape(1, H), be1.reshape(1, H),
                   W2, b2.reshape(1, H), g2.reshape(1, H), be2.reshape(1, H),
                   W3, b3.reshape(1, H))


# speedup vs baseline: 12.6090x; 12.6090x over previous
import jax
import jax.numpy as jnp
from jax import lax
from jax.experimental import pallas as pl
from jax.experimental.pallas import tpu as pltpu
from jax.experimental.pallas import tpu_sc as plsc

N = 10000
E = 320000
D = 128
DE = 16
G = 256
H = 128

NC = 2
NS = 16
NW = NC * NS
EP = E // NW
CW = 80
NCH = EP // CW
EB = 2000
NB = EP // EB
SB = EB // CW
XCH = N // CW


def _sc_segment_sums(x, col, edge_attr, batch2d):
    mesh = plsc.VectorSubcoreMesh(core_axis_name="c", subcore_axis_name="s")

    def body(x_hbm, col_hbm, ea_hbm, batch2_hbm,
             node_out, edge_out,
             acc_node, acc_edge,
             batch_2, col_v, seg_v, ebuf, xbuf, zrow, zrow_e):
        cid = lax.axis_index("c")
        sid = lax.axis_index("s")
        wid = cid * NS + sid

        z16 = jnp.zeros((16,), jnp.float32)
        for r in range(16):
            for o in range(D // 16):
                zrow[r, pl.ds(o * 16, 16)] = z16
            zrow_e[r, pl.ds(0, DE)] = z16
        pltpu.sync_copy(zrow, acc_node.at[pl.ds(sid * 16, 16)])
        pltpu.sync_copy(zrow_e, acc_edge.at[pl.ds(sid * 16, 16)])

        pltpu.sync_copy(batch2_hbm, batch_2)
        pltpu.sync_copy(col_hbm.at[pl.ds(wid * EP, EP)], col_v)

        def seg_body(r, carry):
            for o in range(CW // 16):
                cv = col_v[pl.ds(pl.multiple_of(r * CW + o * 16, 16), 16)]
                sv = plsc.load_gather(batch_2, [cv // CW, cv % CW])
                seg_v[r, pl.ds(o * 16, 16)] = sv
            return carry
        lax.fori_loop(0, NCH, seg_body, 0)

        plsc.subcore_barrier()

        for b in range(NB):
            pltpu.sync_copy(ea_hbm.at[pl.ds(wid * EP + b * EB, EB)], ebuf)

            def sc_body(s, carry):
                off = pl.multiple_of(s * CW, CW)
                pltpu.sync_copy(ebuf.at[pl.ds(off, CW)],
                                acc_edge.at[seg_v.at[b * SB + s]], add=True)
                return carry
            lax.fori_loop(0, SB, sc_body, 0)

        for k in range((XCH + NW - 1) // NW):
            c = wid + NW * k

            @pl.when(c < XCH)
            def _():
                off = pl.multiple_of(c * CW, CW)
                pltpu.sync_copy(x_hbm.at[pl.ds(off, CW)], xbuf)
                pltpu.sync_copy(xbuf, acc_node.at[batch_2.at[c]], add=True)

        plsc.subcore_barrier()

        pltpu.sync_copy(acc_node.at[pl.ds(sid * 16, 16)],
                        node_out.at[cid, pl.ds(sid * 16, 16)])
        pltpu.sync_copy(acc_edge.at[pl.ds(sid * 16, 16)],
                        edge_out.at[cid, pl.ds(sid * 16, 16)])

    f = pl.kernel(
        body,
        out_type=(jax.ShapeDtypeStruct((NC, G, D), jnp.float32),
                  jax.ShapeDtypeStruct((NC, G, DE), jnp.float32)),
        mesh=mesh,
        compiler_params=pltpu.CompilerParams(needs_layout_passes=False,
                                             use_tc_tiling_on_sc=False),
        scratch_types=[
            pltpu.VMEM_SHARED((G, D), jnp.float32),
            pltpu.VMEM_SHARED((G, DE), jnp.float32),
            pltpu.VMEM((XCH, CW), jnp.int32),
            pltpu.VMEM((EP,), jnp.int32),
            pltpu.VMEM((NCH, CW), jnp.int32),
            pltpu.VMEM((EB, DE), jnp.float32),
            pltpu.VMEM((CW, D), jnp.float32),
            pltpu.VMEM((16, D), jnp.float32),
            pltpu.VMEM((16, DE), jnp.float32),
        ],
    )
    return f(x, col, edge_attr, batch2d)


def _tc_mlp(node_part, edge_part, W1n, W1e, b1, g1, be1, W2, b2, g2, be2,
            W3, b3):
    def body(np_ref, ep_ref, w1n_ref, w1e_ref, b1_ref, g1_ref, be1_ref,
             w2_ref, b2_ref, g2_ref, be2_ref, w3_ref, b3_ref, o_ref):
        na = np_ref[0] + np_ref[1]
        ea = ep_ref[0] + ep_ref[1]
        h = jnp.dot(na, w1n_ref[...], preferred_element_type=jnp.float32)
        h = h + jnp.dot(ea, w1e_ref[...], preferred_element_type=jnp.float32)
        h = h + b1_ref[...]
        h = jnp.where(h >= 0, h, 0.01 * h)
        mean = jnp.mean(h, axis=0, keepdims=True)
        var = jnp.mean((h - mean) ** 2, axis=0, keepdims=True)
        h = (h - mean) / jnp.sqrt(var + 1e-5) * g1_ref[...] + be1_ref[...]
        h = jnp.dot(h, w2_ref[...], preferred_element_type=jnp.float32)
        h = h + b2_ref[...]
        h = jnp.where(h >= 0, h, 0.01 * h)
        mean = jnp.mean(h, axis=0, keepdims=True)
        var = jnp.mean((h - mean) ** 2, axis=0, keepdims=True)
        h = (h - mean) / jnp.sqrt(var + 1e-5) * g2_ref[...] + be2_ref[...]
        h = jnp.dot(h, w3_ref[...], preferred_element_type=jnp.float32)
        o_ref[...] = h + b3_ref[...]

    return pl.pallas_call(
        body,
        out_shape=jax.ShapeDtypeStruct((G, H), jnp.float32),
    )(node_part, edge_part, W1n, W1e, b1, g1, be1, W2, b2, g2, be2, W3, b3)


def kernel(x, edge_index, edge_attr, u, batch, W1, b1, g1, be1, W2, b2, g2,
           be2, W3, b3):
    del u
    col = edge_index[1]
    batch2d = batch.reshape(XCH, CW)
    node_part, edge_part = _sc_segment_sums(x, col, edge_attr, batch2d)
    return _tc_mlp(node_part, edge_part,
                   W1[:D], W1[D:],
                   b1.reshape(1, H), g1.reshape(1, H), be1.reshape(1, H),
                   W2, b2.reshape(1, H), g2.reshape(1, H), be2.reshape(1, H),
                   W3, b3.reshape(1, H))


# speedup vs baseline: 13.6765x; 1.0847x over previous
import jax
import jax.numpy as jnp
from jax import lax
from jax.experimental import pallas as pl
from jax.experimental.pallas import tpu as pltpu
from jax.experimental.pallas import tpu_sc as plsc

N = 10000
E = 320000
D = 128
DE = 16
G = 256
H = 128

NC = 2
NS = 16
NW = NC * NS
EP = E // NW
CW = 80
NCH = EP // CW
EB = 2000
NB = EP // EB
SB = EB // CW
XCH = N // CW


def _sc_segment_sums(x, col, edge_attr, batch2d):
    mesh = plsc.VectorSubcoreMesh(core_axis_name="c", subcore_axis_name="s")

    XK = (XCH + NW - 1) // NW

    def body(x_hbm, col_hbm, ea_hbm, batch2_hbm,
             node_out, edge_out,
             acc_node, acc_edge,
             batch_2, col_v, seg_v, ebuf, xbuf, zrow, zrow_e,
             blsem, clsem, elsem, essem, xlsem, xssem):
        cid = lax.axis_index("c")
        sid = lax.axis_index("s")
        wid = cid * NS + sid

        def eload(b, p, sem):
            return pltpu.make_async_copy(
                ea_hbm.at[pl.ds(wid * EP + b * EB, EB)], ebuf.at[p], sem)

        def xload(k, p, sem):
            c = wid + NW * k
            off = pl.multiple_of(c * CW, CW)
            return pltpu.make_async_copy(x_hbm.at[pl.ds(off, CW)],
                                         xbuf.at[p], sem)

        pltpu.async_copy(batch2_hbm, batch_2, blsem)
        pltpu.async_copy(col_hbm.at[pl.ds(wid * EP, EP)], col_v, clsem)
        for b in range(2):
            eload(b, b, elsem.at[b]).start()
        for k in range(2):
            xload(k, k, xlsem.at[k]).start()

        z16 = jnp.zeros((16,), jnp.float32)
        for r in range(16):
            for o in range(D // 16):
                zrow[r, pl.ds(o * 16, 16)] = z16
            zrow_e[r, pl.ds(0, DE)] = z16
        pltpu.sync_copy(zrow, acc_node.at[pl.ds(sid * 16, 16)])
        pltpu.sync_copy(zrow_e, acc_edge.at[pl.ds(sid * 16, 16)])

        pltpu.make_async_copy(batch2_hbm, batch_2, blsem).wait()
        pltpu.make_async_copy(col_hbm.at[pl.ds(wid * EP, EP)], col_v,
                              clsem).wait()

        def seg_body(r, carry):
            for o in range(CW // 16):
                cv = col_v[pl.ds(pl.multiple_of(r * CW + o * 16, 16), 16)]
                sv = plsc.load_gather(batch_2, [cv // CW, cv % CW])
                seg_v[r, pl.ds(o * 16, 16)] = sv
            return carry
        lax.fori_loop(0, NCH, seg_body, 0)

        plsc.subcore_barrier()

        for b in range(NB):
            p = b % 2
            eload(b, p, elsem.at[p]).wait()

            def sc_fire(s, carry):
                off = pl.multiple_of(s * CW, CW)
                pltpu.async_copy(ebuf.at[p, pl.ds(off, CW)],
                                 acc_edge.at[seg_v.at[b * SB + s]],
                                 essem.at[p], add=True)
                return carry
            lax.fori_loop(0, SB, sc_fire, 0)

            def sc_drain(s, carry):
                off = pl.multiple_of(s * CW, CW)
                pltpu.make_async_copy(ebuf.at[p, pl.ds(off, CW)],
                                      acc_edge.at[seg_v.at[b * SB + s]],
                                      essem.at[p]).wait()
                return carry
            lax.fori_loop(0, SB, sc_drain, 0)
            if b + 2 < NB:
                eload(b + 2, p, elsem.at[p]).start()

        for k in range(XK):
            p = k % 2
            c = wid + NW * k

            @pl.when(c < XCH)
            def _():
                xload(k, p, xlsem.at[p]).wait()
                pltpu.async_copy(xbuf.at[p], acc_node.at[batch_2.at[c]],
                                 xssem.at[p], add=True)
                pltpu.make_async_copy(xbuf.at[p], acc_node.at[batch_2.at[c]],
                                      xssem.at[p]).wait()
            if k + 2 < XK:

                @pl.when(wid + NW * (k + 2) < XCH)
                def _():
                    xload(k + 2, p, xlsem.at[p]).start()

        plsc.subcore_barrier()

        pltpu.sync_copy(acc_node.at[pl.ds(sid * 16, 16)],
                        node_out.at[cid, pl.ds(sid * 16, 16)])
        pltpu.sync_copy(acc_edge.at[pl.ds(sid * 16, 16)],
                        edge_out.at[cid, pl.ds(sid * 16, 16)])

    f = pl.kernel(
        body,
        out_type=(jax.ShapeDtypeStruct((NC, G, D), jnp.float32),
                  jax.ShapeDtypeStruct((NC, G, DE), jnp.float32)),
        mesh=mesh,
        compiler_params=pltpu.CompilerParams(needs_layout_passes=False,
                                             use_tc_tiling_on_sc=False),
        scratch_types=[
            pltpu.VMEM_SHARED((G, D), jnp.float32),
            pltpu.VMEM_SHARED((G, DE), jnp.float32),
            pltpu.VMEM((XCH, CW), jnp.int32),
            pltpu.VMEM((EP,), jnp.int32),
            pltpu.VMEM((NCH, CW), jnp.int32),
            pltpu.VMEM((2, EB, DE), jnp.float32),
            pltpu.VMEM((2, CW, D), jnp.float32),
            pltpu.VMEM((16, D), jnp.float32),
            pltpu.VMEM((16, DE), jnp.float32),
            pltpu.SemaphoreType.DMA,
            pltpu.SemaphoreType.DMA,
            pltpu.SemaphoreType.DMA((2,)),
            pltpu.SemaphoreType.DMA((2,)),
            pltpu.SemaphoreType.DMA((2,)),
            pltpu.SemaphoreType.DMA((2,)),
        ],
    )
    return f(x, col, edge_attr, batch2d)


def _tc_mlp(node_part, edge_part, W1n, W1e, b1, g1, be1, W2, b2, g2, be2,
            W3, b3):
    def body(np_ref, ep_ref, w1n_ref, w1e_ref, b1_ref, g1_ref, be1_ref,
             w2_ref, b2_ref, g2_ref, be2_ref, w3_ref, b3_ref, o_ref):
        na = np_ref[0] + np_ref[1]
        ea = ep_ref[0] + ep_ref[1]
        h = jnp.dot(na, w1n_ref[...], preferred_element_type=jnp.float32)
        h = h + jnp.dot(ea, w1e_ref[...], preferred_element_type=jnp.float32)
        h = h + b1_ref[...]
        h = jnp.where(h >= 0, h, 0.01 * h)
        mean = jnp.mean(h, axis=0, keepdims=True)
        var = jnp.mean((h - mean) ** 2, axis=0, keepdims=True)
        h = (h - mean) / jnp.sqrt(var + 1e-5) * g1_ref[...] + be1_ref[...]
        h = jnp.dot(h, w2_ref[...], preferred_element_type=jnp.float32)
        h = h + b2_ref[...]
        h = jnp.where(h >= 0, h, 0.01 * h)
        mean = jnp.mean(h, axis=0, keepdims=True)
        var = jnp.mean((h - mean) ** 2, axis=0, keepdims=True)
        h = (h - mean) / jnp.sqrt(var + 1e-5) * g2_ref[...] + be2_ref[...]
        h = jnp.dot(h, w3_ref[...], preferred_element_type=jnp.float32)
        o_ref[...] = h + b3_ref[...]

    return pl.pallas_call(
        body,
        out_shape=jax.ShapeDtypeStruct((G, H), jnp.float32),
    )(node_part, edge_part, W1n, W1e, b1, g1, be1, W2, b2, g2, be2, W3, b3)


def kernel(x, edge_index, edge_attr, u, batch, W1, b1, g1, be1, W2, b2, g2,
           be2, W3, b3):
    del u
    col = edge_index[1]
    batch2d = batch.reshape(XCH, CW)
    node_part, edge_part = _sc_segment_sums(x, col, edge_attr, batch2d)
    return _tc_mlp(node_part, edge_part,
                   W1[:D], W1[D:],
                   b1.reshape(1, H), g1.reshape(1, H), be1.reshape(1, H),
                   W2, b2.reshape(1, H), g2.reshape(1, H), be2.reshape(1, H),
                   W3, b3.reshape(1, H))
